# baseline (speedup 1.0000x reference)
import jax
import jax.numpy as jnp
from jax.experimental import pallas as pl
from jax.experimental.pallas import tpu as pltpu

N = 10000
NFEAT = 128
H1 = 64
H2 = 32

ROWS_BLK = 400
NB = N // ROWS_BLK
NS = 5
HR = ROWS_BLK // NS

_PARAMS = pltpu.CompilerParams(
    dimension_semantics=("arbitrary",),
    vmem_limit_bytes=64 * 1024 * 1024,
)


def _gcn_body(x_ref, *refs):
    adj_refs = refs[:NS]
    w1_ref, b1_ref, w2_ref, b2_ref, o_ref, s1_ref, s2_ref = refs[NS:]
    g = pl.program_id(0)

    @pl.when(g == 0)
    def _():
        s1_ref[...] = jnp.dot(
            x_ref[...], w1_ref[...], preferred_element_type=jnp.float32
        )

    @pl.when((g >= 1) & (g <= NB))
    def _():
        i = NB - g
        for s in range(NS):
            h = jnp.dot(
                adj_refs[s][...], s1_ref[...], preferred_element_type=jnp.float32
            )
            h = jnp.maximum(h + b1_ref[...], 0.0)
            s2_ref[pl.ds(i * ROWS_BLK + s * HR, HR), :] = jnp.dot(
                h, w2_ref[...], preferred_element_type=jnp.float32
            )

    @pl.when(g > NB)
    def _():
        for s in range(NS):
            o_ref[s * HR : (s + 1) * HR, :] = (
                jnp.dot(
                    adj_refs[s][...], s2_ref[...], preferred_element_type=jnp.float32
                )
                + b2_ref[...]
            )


def _adj_row(g):
    p1 = NB - g
    p2 = g - 1 - NB
    return jnp.where(g == 0, NB - 1, jnp.where(g <= NB, p1, p2))


def _make_adj_index(s):
    def _idx(g):
        return (NS * _adj_row(g) + s, 0)

    return _idx


def _out_index(g):
    return (jnp.maximum(g - 1 - NB, 0), 0)


@jax.jit
def _gcn(x, adj, W1, b1, W2, b2):
    b1r = b1.reshape(1, H1)
    b2r = b2.reshape(1, H2)

    out = pl.pallas_call(
        _gcn_body,
        grid=(1 + 2 * NB,),
        in_specs=[
            pl.BlockSpec((N, NFEAT), lambda g: (0, 0)),
            *[pl.BlockSpec((HR, N), _make_adj_index(s)) for s in range(NS)],
            pl.BlockSpec((NFEAT, H1), lambda g: (0, 0)),
            pl.BlockSpec((1, H1), lambda g: (0, 0)),
            pl.BlockSpec((H1, H2), lambda g: (0, 0)),
            pl.BlockSpec((1, H2), lambda g: (0, 0)),
        ],
        out_specs=pl.BlockSpec((ROWS_BLK, H2), _out_index),
        out_shape=jax.ShapeDtypeStruct((N, H2), jnp.float32),
        scratch_shapes=[
            pltpu.VMEM((N, H1), jnp.float32),
            pltpu.VMEM((N, H2), jnp.float32),
        ],
        compiler_params=_PARAMS,
    )(x, *([adj] * NS), W1, b1r, W2, b2r)

    return out


def kernel(x, adj, W1, b1, W2, b2):
    return _gcn(x, adj, W1, b1, W2, b2)


# speedup vs baseline: 1.0135x; 1.0135x over previous
import functools

import jax
import jax.numpy as jnp
from jax.experimental import pallas as pl
from jax.experimental.pallas import tpu as pltpu

N = 10000
NFEAT = 128
H1 = 64
H2 = 32

CH = 200
NCH = N // CH
K = 5

_PARAMS = pltpu.CompilerParams(
    dimension_semantics=(),
    vmem_limit_bytes=64 * 1024 * 1024,
)


def _fetch(adj_ref, abuf_ref, sem_ref, c, slot):
    pltpu.make_async_copy(
        adj_ref.at[pl.ds(c * CH, CH), :], abuf_ref.at[slot], sem_ref.at[slot]
    ).start()


def _wait(adj_ref, abuf_ref, sem_ref, c, slot):
    pltpu.make_async_copy(
        adj_ref.at[pl.ds(c * CH, CH), :], abuf_ref.at[slot], sem_ref.at[slot]
    ).wait()


def _gcn_body(
    x_ref, adj_ref, w1_ref, b1_ref, w2_ref, b2_ref, o_ref, s1_ref, s2_ref, abuf_ref, sem_ref
):
    for k in range(K):
        c0 = NCH - 1 - k
        _fetch(adj_ref, abuf_ref, sem_ref, c0, c0 % K)

    s1_ref[...] = jnp.dot(x_ref[...], w1_ref[...], preferred_element_type=jnp.float32)

    def p1_body(it, _):
        c = NCH - 1 - it
        slot = jax.lax.rem(c, K)
        _wait(adj_ref, abuf_ref, sem_ref, c, slot)
        h = jnp.dot(abuf_ref[slot], s1_ref[...], preferred_element_type=jnp.float32)
        h = jnp.maximum(h + b1_ref[...], 0.0)
        s2_ref[pl.ds(c * CH, CH), :] = jnp.dot(
            h, w2_ref[...], preferred_element_type=jnp.float32
        )

        @pl.when(c >= K)
        def _():
            _fetch(adj_ref, abuf_ref, sem_ref, c - K, slot)

        return 0

    jax.lax.fori_loop(0, NCH, p1_body, 0)

    def p2_body(c, _):
        slot = jax.lax.rem(c, K)

        @pl.when(c >= K)
        def _():
            _wait(adj_ref, abuf_ref, sem_ref, c, slot)

        o_ref[pl.ds(c * CH, CH), :] = (
            jnp.dot(abuf_ref[slot], s2_ref[...], preferred_element_type=jnp.float32)
            + b2_ref[...]
        )

        @pl.when(c + K < NCH)
        def _():
            _fetch(adj_ref, abuf_ref, sem_ref, c + K, slot)

        return 0

    jax.lax.fori_loop(0, NCH, p2_body, 0)


@jax.jit
def _gcn(x, adj, W1, b1, W2, b2):
    b1r = b1.reshape(1, H1)
    b2r = b2.reshape(1, H2)

    out = pl.pallas_call(
        _gcn_body,
        in_specs=[
            pl.BlockSpec(memory_space=pltpu.MemorySpace.VMEM),
            pl.BlockSpec(memory_space=pl.ANY),
            pl.BlockSpec(memory_space=pltpu.MemorySpace.VMEM),
            pl.BlockSpec(memory_space=pltpu.MemorySpace.VMEM),
            pl.BlockSpec(memory_space=pltpu.MemorySpace.VMEM),
            pl.BlockSpec(memory_space=pltpu.MemorySpace.VMEM),
        ],
        out_specs=pl.BlockSpec(memory_space=pltpu.MemorySpace.VMEM),
        out_shape=jax.ShapeDtypeStruct((N, H2), jnp.float32),
        scratch_shapes=[
            pltpu.VMEM((N, H1), jnp.float32),
            pltpu.VMEM((N, H2), jnp.float32),
            pltpu.VMEM((K, CH, N), jnp.float32),
            pltpu.SemaphoreType.DMA((K,)),
        ],
        compiler_params=_PARAMS,
    )(x, adj, W1, b1r, W2, b2r)

    return out


def kernel(x, adj, W1, b1, W2, b2):
    return _gcn(x, adj, W1, b1, W2, b2)
